# Initial kernel scaffold; baseline (speedup 1.0000x reference)
import jax
import jax.numpy as jnp
from jax.experimental import pallas as pl


def kernel(x, emb):
    raise NotImplementedError("write your pallas kernel here")


# speedup vs baseline: 1.1913x; 1.1913x over previous
import functools

import jax
import jax.numpy as jnp
from jax import lax
from jax.experimental import pallas as pl
from jax.experimental.pallas import tpu as pltpu
from jax.experimental.pallas import tpu_sc as plsc

_DIM = 256
_CODES = 8192
_TBLK = 256

_NC = 2
_NS = 16
_NW = _NC * _NS
_ICHUNK = 128


def _dist_argmin_body(nelem, x_ref, emb_ref, idx_ref, loss_ref,
                      e2_ref, acc_ref):
    step = pl.program_id(0)

    @pl.when(step == 0)
    def _init():
        e = emb_ref[...]
        e2_ref[...] = jnp.sum(e * e, axis=1, keepdims=True).reshape(1, _CODES)
        acc_ref[0] = 0.0

    xt = jnp.transpose(x_ref[0])
    mm = lax.dot_general(
        xt, emb_ref[...], (((1,), (1,)), ((), ())),
        preferred_element_type=jnp.float32)
    f2 = jnp.sum(xt * xt, axis=1, keepdims=True)
    dist = (f2 - 2.0 * mm) + e2_ref[...]
    minv = jnp.min(dist, axis=1, keepdims=True)
    codes = lax.broadcasted_iota(jnp.int32, dist.shape, 1)
    idx = jnp.min(jnp.where(dist == minv, codes, _CODES), axis=1)
    idx_ref[0, 0, :] = idx
    acc_ref[0] += jnp.sum(minv)

    @pl.when(step == pl.num_programs(0) - 1)
    def _done():
        m = acc_ref[0] / jnp.float32(nelem)
        loss_ref[0, 0] = m + m


def _gather_body(rows_per_tile, emb_hbm, idx_hbm, out_hbm, idx_v, rows_v,
                 sem):
    wid = lax.axis_index("s") * _NC + lax.axis_index("c")
    base = wid * rows_per_tile
    nchunk = rows_per_tile // _ICHUNK
    for k in range(nchunk):
        pltpu.sync_copy(idx_hbm.at[pl.ds(base + k * _ICHUNK, _ICHUNK)],
                        idx_v.at[k])
    copies = [pltpu.async_copy(emb_hbm.at[idx_v.at[k]], rows_v.at[k], sem)
              for k in range(nchunk)]
    for c in copies:
        c.wait()
    for k in range(nchunk):
        pltpu.sync_copy(rows_v.at[k],
                        out_hbm.at[pl.ds(base + k * _ICHUNK, _ICHUNK)])


def _finish_body(x_ref, q_ref, out_ref):
    xb = x_ref[0]
    qt = jnp.transpose(q_ref[...])
    out_ref[0] = xb + (qt - xb)


def kernel(x, emb):
    B, C, T = x.shape
    tpb = T // _TBLK
    nsteps = B * tpb

    idx3, loss = pl.pallas_call(
        functools.partial(_dist_argmin_body, B * C * T),
        grid=(nsteps,),
        in_specs=[
            pl.BlockSpec((1, C, _TBLK), lambda i: (i // tpb, 0, i % tpb)),
            pl.BlockSpec((_CODES, C), lambda i: (0, 0)),
        ],
        out_specs=[
            pl.BlockSpec((1, 1, _TBLK), lambda i: (i, 0, 0)),
            pl.BlockSpec(memory_space=pltpu.SMEM),
        ],
        out_shape=[
            jax.ShapeDtypeStruct((nsteps, 1, _TBLK), jnp.int32),
            jax.ShapeDtypeStruct((1, 1), jnp.float32),
        ],
        scratch_shapes=[
            pltpu.VMEM((1, _CODES), jnp.float32),
            pltpu.SMEM((1,), jnp.float32),
        ],
        compiler_params=pltpu.CompilerParams(
            dimension_semantics=("arbitrary",)),
    )(x, emb)

    idx = idx3.reshape(-1)

    mesh = plsc.VectorSubcoreMesh(core_axis_name="c", subcore_axis_name="s")
    rows_per_tile = (B * T) // _NW
    nchunk = rows_per_tile // _ICHUNK
    gather = pl.kernel(
        functools.partial(_gather_body, rows_per_tile),
        mesh=mesh,
        out_type=jax.ShapeDtypeStruct((B * T, C), jnp.float32),
        scratch_types=[
            pltpu.VMEM((nchunk, _ICHUNK), jnp.int32),
            pltpu.VMEM((nchunk, _ICHUNK, C), jnp.float32),
            pltpu.SemaphoreType.DMA,
        ],
    )
    quant_flat = gather(emb, idx)

    quantized = pl.pallas_call(
        _finish_body,
        grid=(nsteps,),
        in_specs=[
            pl.BlockSpec((1, C, _TBLK), lambda i: (i // tpb, 0, i % tpb)),
            pl.BlockSpec((_TBLK, C), lambda i: (i, 0)),
        ],
        out_specs=pl.BlockSpec((1, C, _TBLK),
                               lambda i: (i // tpb, 0, i % tpb)),
        out_shape=jax.ShapeDtypeStruct((B, C, T), jnp.float32),
        compiler_params=pltpu.CompilerParams(
            dimension_semantics=("arbitrary",)),
    )(x, quant_flat)

    return (quantized, loss[0, 0])


# speedup vs baseline: 1.2935x; 1.0858x over previous
import functools

import jax
import jax.numpy as jnp
from jax import lax
from jax.experimental import pallas as pl
from jax.experimental.pallas import tpu as pltpu
from jax.experimental.pallas import tpu_sc as plsc

_DIM = 256
_CODES = 8192
_TBLK = 1024

_NC = 2
_NS = 16
_NW = _NC * _NS
_ICHUNK = 128


def _dist_argmin_body(nelem, x_ref, emb_ref, idx_ref, loss_ref,
                      e2_ref, acc_ref):
    step = pl.program_id(0)

    @pl.when(step == 0)
    def _init():
        e = emb_ref[...]
        e2_ref[...] = jnp.sum(e * e, axis=1, keepdims=True).reshape(1, _CODES)
        acc_ref[0] = 0.0

    xt = jnp.transpose(x_ref[0])
    mm = lax.dot_general(
        xt, emb_ref[...], (((1,), (1,)), ((), ())),
        preferred_element_type=jnp.float32)
    f2 = jnp.sum(xt * xt, axis=1, keepdims=True)
    dist = (f2 - 2.0 * mm) + e2_ref[...]
    minv = jnp.min(dist, axis=1)
    idx = jnp.argmin(dist, axis=1)
    idx_ref[0, 0, :] = idx
    acc_ref[0] += jnp.sum(minv)

    @pl.when(step == pl.num_programs(0) - 1)
    def _done():
        m = acc_ref[0] / jnp.float32(nelem)
        loss_ref[0, 0] = m + m


def _gather_body(rows_per_tile, emb_hbm, idx_hbm, out_hbm, idx_v, rows_v,
                 sem):
    wid = lax.axis_index("s") * _NC + lax.axis_index("c")
    base = wid * rows_per_tile
    nchunk = rows_per_tile // _ICHUNK
    for k in range(nchunk):
        pltpu.sync_copy(idx_hbm.at[pl.ds(base + k * _ICHUNK, _ICHUNK)],
                        idx_v.at[k])
    copies = [pltpu.async_copy(emb_hbm.at[idx_v.at[k]], rows_v.at[k], sem)
              for k in range(nchunk)]
    for c in copies:
        c.wait()
    for k in range(nchunk):
        pltpu.sync_copy(rows_v.at[k],
                        out_hbm.at[pl.ds(base + k * _ICHUNK, _ICHUNK)])


def _finish_body(x_ref, q_ref, out_ref):
    xb = x_ref[0]
    qt = jnp.transpose(q_ref[...])
    out_ref[0] = xb + (qt - xb)


def kernel(x, emb):
    B, C, T = x.shape
    tpb = T // _TBLK
    nsteps = B * tpb

    idx3, loss = pl.pallas_call(
        functools.partial(_dist_argmin_body, B * C * T),
        grid=(nsteps,),
        in_specs=[
            pl.BlockSpec((1, C, _TBLK), lambda i: (i // tpb, 0, i % tpb)),
            pl.BlockSpec((_CODES, C), lambda i: (0, 0)),
        ],
        out_specs=[
            pl.BlockSpec((1, 1, _TBLK), lambda i: (i, 0, 0)),
            pl.BlockSpec(memory_space=pltpu.SMEM),
        ],
        out_shape=[
            jax.ShapeDtypeStruct((nsteps, 1, _TBLK), jnp.int32),
            jax.ShapeDtypeStruct((1, 1), jnp.float32),
        ],
        scratch_shapes=[
            pltpu.VMEM((1, _CODES), jnp.float32),
            pltpu.SMEM((1,), jnp.float32),
        ],
        compiler_params=pltpu.CompilerParams(
            dimension_semantics=("arbitrary",)),
    )(x, emb)

    idx = idx3.reshape(-1)

    mesh = plsc.VectorSubcoreMesh(core_axis_name="c", subcore_axis_name="s")
    rows_per_tile = (B * T) // _NW
    nchunk = rows_per_tile // _ICHUNK
    gather = pl.kernel(
        functools.partial(_gather_body, rows_per_tile),
        mesh=mesh,
        out_type=jax.ShapeDtypeStruct((B * T, C), jnp.float32),
        scratch_types=[
            pltpu.VMEM((nchunk, _ICHUNK), jnp.int32),
            pltpu.VMEM((nchunk, _ICHUNK, C), jnp.float32),
            pltpu.SemaphoreType.DMA,
        ],
    )
    quant_flat = gather(emb, idx)

    quantized = pl.pallas_call(
        _finish_body,
        grid=(nsteps,),
        in_specs=[
            pl.BlockSpec((1, C, _TBLK), lambda i: (i // tpb, 0, i % tpb)),
            pl.BlockSpec((_TBLK, C), lambda i: (i, 0)),
        ],
        out_specs=pl.BlockSpec((1, C, _TBLK),
                               lambda i: (i // tpb, 0, i % tpb)),
        out_shape=jax.ShapeDtypeStruct((B, C, T), jnp.float32),
        compiler_params=pltpu.CompilerParams(
            dimension_semantics=("arbitrary",)),
    )(x, quant_flat)

    return (quantized, loss[0, 0])
